# baseline (speedup 1.0000x reference)
import jax
import jax.numpy as jnp
from jax import lax
from jax.experimental import pallas as pl
from jax.experimental.pallas import tpu as pltpu
from jax.experimental.pallas import tpu_sc as plsc

_EMBED_DIM = 64
_PHASE_DIM = 16
_NW = 32
_CHUNK = 128


def _make_sc_gather(n_total):
    per_w = n_total // _NW
    nch = per_w // _CHUNK
    mesh = plsc.VectorSubcoreMesh(core_axis_name="c", subcore_axis_name="s")

    def body(x_hbm, real_hbm, imag_hbm, real_out, imag_out,
             idx_v, rrows, irows, sem_r, sem_i):
        info = plsc.get_sparse_core_info()
        wid = lax.axis_index("s") * info.num_cores + lax.axis_index("c")
        pltpu.sync_copy(x_hbm.at[pl.ds(wid * nch, nch), :], idx_v)
        base = wid * per_w

        def step(j, carry):
            cr = pltpu.async_copy(real_hbm.at[idx_v.at[j]], rrows, sem_r)
            ci = pltpu.async_copy(imag_hbm.at[idx_v.at[j]], irows, sem_i)
            cr.wait()
            ci.wait()
            off = base + j * _CHUNK
            pltpu.sync_copy(rrows, real_out.at[pl.ds(off, _CHUNK), :])
            pltpu.sync_copy(irows, imag_out.at[pl.ds(off, _CHUNK), :])
            return carry

        lax.fori_loop(0, nch, step, 0)

    return pl.kernel(
        body,
        out_type=(
            jax.ShapeDtypeStruct((n_total, _EMBED_DIM), jnp.float32),
            jax.ShapeDtypeStruct((n_total, _PHASE_DIM), jnp.float32),
        ),
        mesh=mesh,
        scratch_types=[
            pltpu.VMEM((nch, _CHUNK), jnp.int32),
            pltpu.VMEM((_CHUNK, _EMBED_DIM), jnp.float32),
            pltpu.VMEM((_CHUNK, _PHASE_DIM), jnp.float32),
            pltpu.SemaphoreType.DMA,
            pltpu.SemaphoreType.DMA,
        ],
        compiler_params=pltpu.CompilerParams(use_tc_tiling_on_sc=False),
    )


def kernel(x, real_table, imag_table):
    b, h = x.shape
    n = b * h
    x2 = x.reshape(n // _CHUNK, _CHUNK).astype(jnp.int32)
    real_flat, imag_flat = _make_sc_gather(n)(x2, real_table, imag_table)
    return (real_flat.reshape(b, h, _EMBED_DIM),
            imag_flat.reshape(b, h, _PHASE_DIM))


# speedup vs baseline: 1.0770x; 1.0770x over previous
import jax
import jax.numpy as jnp
from jax import lax
from jax.experimental import pallas as pl
from jax.experimental.pallas import tpu as pltpu
from jax.experimental.pallas import tpu_sc as plsc

_EMBED_DIM = 64
_PHASE_DIM = 16
_NW = 32
_CHUNK = 128
_K = 2
_G = _K * _CHUNK
_NBUF = 4


def _make_sc_gather(n_total):
    per_w = n_total // _NW
    nch = per_w // _CHUNK
    ng = per_w // _G
    mesh = plsc.VectorSubcoreMesh(core_axis_name="c", subcore_axis_name="s")

    def body(x_hbm, real_hbm, imag_hbm, real_out, imag_out, idx_v, *scr):
        bufr = scr[0:_NBUF]
        bufi = scr[_NBUF:2 * _NBUF]
        gsem = scr[2 * _NBUF:3 * _NBUF]
        osem = scr[3 * _NBUF:4 * _NBUF]

        info = plsc.get_sparse_core_info()
        wid = lax.axis_index("s") * info.num_cores + lax.axis_index("c")
        pltpu.sync_copy(x_hbm.at[pl.ds(wid * nch, nch), :], idx_v)
        base = wid * per_w

        def fire_gather(g, b):
            for k in range(_K):
                row = g * _K + k
                pltpu.async_copy(real_hbm.at[idx_v.at[row]],
                                 bufr[b].at[pl.ds(k * _CHUNK, _CHUNK), :],
                                 gsem[b])
                pltpu.async_copy(imag_hbm.at[idx_v.at[row]],
                                 bufi[b].at[pl.ds(k * _CHUNK, _CHUNK), :],
                                 gsem[b])

        def drain_gather(b):
            pltpu.make_async_copy(real_hbm.at[pl.ds(0, _G)], bufr[b],
                                  gsem[b]).wait()
            pltpu.make_async_copy(imag_hbm.at[pl.ds(0, _G)], bufi[b],
                                  gsem[b]).wait()

        def fire_out(g, b):
            off = base + g * _G
            pltpu.async_copy(bufr[b], real_out.at[pl.ds(off, _G), :], osem[b])
            pltpu.async_copy(bufi[b], imag_out.at[pl.ds(off, _G), :], osem[b])

        def drain_out(b):
            pltpu.make_async_copy(bufr[b], real_out.at[pl.ds(0, _G), :],
                                  osem[b]).wait()
            pltpu.make_async_copy(bufi[b], imag_out.at[pl.ds(0, _G), :],
                                  osem[b]).wait()

        for b in range(_NBUF - 1):
            fire_gather(b, b)

        def outer(g0, carry):
            for b in range(_NBUF):
                g = g0 * _NBUF + b
                bn = (b + _NBUF - 1) % _NBUF

                @pl.when(g + _NBUF - 1 < ng)
                def _():
                    @pl.when(g >= 1)
                    def _():
                        drain_out(bn)
                    fire_gather(g + _NBUF - 1, bn)

                drain_gather(b)
                fire_out(g, b)
            return carry

        lax.fori_loop(0, ng // _NBUF, outer, 0)
        for b in range(_NBUF):
            drain_out(b)

    return pl.kernel(
        body,
        out_type=(
            jax.ShapeDtypeStruct((n_total, _EMBED_DIM), jnp.float32),
            jax.ShapeDtypeStruct((n_total, _PHASE_DIM), jnp.float32),
        ),
        mesh=mesh,
        scratch_types=(
            [pltpu.VMEM((nch, _CHUNK), jnp.int32)]
            + [pltpu.VMEM((_G, _EMBED_DIM), jnp.float32)] * _NBUF
            + [pltpu.VMEM((_G, _PHASE_DIM), jnp.float32)] * _NBUF
            + [pltpu.SemaphoreType.DMA] * (2 * _NBUF)
        ),
        compiler_params=pltpu.CompilerParams(use_tc_tiling_on_sc=False),
    )


def kernel(x, real_table, imag_table):
    b, h = x.shape
    n = b * h
    x2 = x.reshape(n // _CHUNK, _CHUNK).astype(jnp.int32)
    real_flat, imag_flat = _make_sc_gather(n)(x2, real_table, imag_table)
    return (real_flat.reshape(b, h, _EMBED_DIM),
            imag_flat.reshape(b, h, _PHASE_DIM))
